# Initial kernel scaffold; baseline (speedup 1.0000x reference)
import jax
import jax.numpy as jnp
from jax.experimental import pallas as pl


def kernel(tag1, tag2, ind1, ind2, mask):
    raise NotImplementedError("write your pallas kernel here")


# speedup vs baseline: 1.1104x; 1.1104x over previous
import functools

import jax
import jax.numpy as jnp
from jax import lax
from jax.experimental import pallas as pl
from jax.experimental.pallas import tpu as pltpu
from jax.experimental.pallas import tpu_sc as plsc

NC, NS, L = 2, 16, 16
B = 16
K = 500
KP = 512
NCHUNK = KP // L
HW = 128 * 128
NACC = 4


def _tec_body(tag1_hbm, tag2_hbm, ind1_hbm, ind2_hbm, mask_hbm, out_hbm,
              tagrow, ind_v, mask_v, t0_v, t1_v, mcomp, row_v):
    c = lax.axis_index("c")
    s = lax.axis_index("s")
    wid = s * NC + c

    @pl.when(wid < B)
    def _():
        b = wid

        pltpu.sync_copy(mask_hbm.at[b], mask_v)
        pltpu.sync_copy(ind1_hbm.at[b], ind_v)
        pltpu.sync_copy(tag1_hbm.at[b], tagrow)
        for jc in range(NCHUNK):
            sl = pl.ds(jc * L, L)
            t0_v[sl] = plsc.load_gather(tagrow, [ind_v[sl]])
        pltpu.sync_copy(ind2_hbm.at[b], ind_v)
        pltpu.sync_copy(tag2_hbm.at[b], tagrow)
        for jc in range(NCHUNK):
            sl = pl.ds(jc * L, L)
            t1_v[sl] = plsc.load_gather(tagrow, [ind_v[sl]])

        inf_v = jnp.full((L,), jnp.inf, jnp.float32)
        for jc in range(NCHUNK):
            mcomp[pl.ds(jc * L, L)] = inf_v
        offset = jnp.int32(0)
        psum = jnp.zeros((L,), jnp.float32)
        for jc in range(NCHUNK):
            sl = pl.ds(jc * L, L)
            m = mask_v[sl]
            mb = m > 0
            a = t0_v[sl]
            bb = t1_v[sl]
            mean = (a + bb) * 0.5
            d = a - bb
            psum = psum + jnp.where(mb, d * d, 0.0)
            pos = jnp.maximum(offset + lax.cumsum(m, axis=0) - 1, 0)
            plsc.store_scatter(mcomp, [pos], mean, mask=mb)
            offset = offset + jnp.sum(m)
        n = offset
        pullsq = jnp.sum(psum)

        mj = [mcomp[pl.ds(jc * L, L)] for jc in range(NCHUNK)]
        zero = jnp.zeros((L,), jnp.float32)

        def body(i, accs):
            accl = list(accs)
            miv = plsc.load_gather(mcomp, [jnp.full((L,), i, jnp.int32)])
            for jc in range(NCHUNK):
                r = jnp.maximum(1.0 - jnp.abs(mj[jc] - miv), 0.0)
                accl[jc % NACC] = accl[jc % NACC] + r
            return tuple(accl)

        accs = lax.fori_loop(0, n, body, (zero,) * NACC)
        S = jnp.sum(sum(accs[1:], accs[0]))

        nfv = jnp.full((L,), n.astype(jnp.float32))
        pullv = jnp.full((L,), pullsq) / (2.0 * (nfv + 1e-4))
        pushv = (jnp.full((L,), S) - nfv * nfv / (nfv + 1e-4)) / (
            (nfv - 1.0) * nfv + 1e-4)
        lane = lax.iota(jnp.int32, L)
        row_v[...] = jnp.where(lane == 0, pullv,
                               jnp.where(lane == 1, pushv, 0.0))
        pltpu.sync_copy(row_v, out_hbm.at[b])


@functools.partial(
    pl.kernel,
    out_type=jax.ShapeDtypeStruct((B, L), jnp.float32),
    mesh=plsc.VectorSubcoreMesh(core_axis_name="c", subcore_axis_name="s"),
    compiler_params=pltpu.CompilerParams(needs_layout_passes=False),
    scratch_types=[
        pltpu.VMEM((HW,), jnp.float32),
        pltpu.VMEM((KP,), jnp.int32),
        pltpu.VMEM((KP,), jnp.int32),
        pltpu.VMEM((KP,), jnp.float32),
        pltpu.VMEM((KP,), jnp.float32),
        pltpu.VMEM((KP,), jnp.float32),
        pltpu.VMEM((L,), jnp.float32),
    ],
)
def _tag_loss_sc(tag1_hbm, tag2_hbm, ind1_hbm, ind2_hbm, mask_hbm, out_hbm,
                 *scratch):
    _tec_body(tag1_hbm, tag2_hbm, ind1_hbm, ind2_hbm, mask_hbm, out_hbm,
              *scratch)


@jax.jit
def kernel(tag1, tag2, ind1, ind2, mask):
    tag1f = tag1.reshape(B, HW)
    tag2f = tag2.reshape(B, HW)
    pad = ((0, 0), (0, KP - K))
    ind1p = jnp.pad(ind1.astype(jnp.int32), pad)
    ind2p = jnp.pad(ind2.astype(jnp.int32), pad)
    maskp = jnp.pad(mask.astype(jnp.int32), pad)
    out = _tag_loss_sc(tag1f, tag2f, ind1p, ind2p, maskp)
    return (out[:, 0].sum(), out[:, 1].sum())


# speedup vs baseline: 1.3136x; 1.1830x over previous
import functools

import jax
import jax.numpy as jnp
from jax import lax
from jax.experimental import pallas as pl
from jax.experimental.pallas import tpu as pltpu
from jax.experimental.pallas import tpu_sc as plsc

NC, NS, L = 2, 16, 16
B = 16
K = 500
KP = 512
NCHUNK = KP // L
HW = 128 * 128
NACC = 4


def _tec_body(tag1_hbm, tag2_hbm, ind1_hbm, ind2_hbm, mask_hbm, out_hbm,
              tagrow, ind_v, mask_v, tmine, tother, mcomp, row_v, shared):
    c = lax.axis_index("c")
    s = lax.axis_index("s")
    b = c * (B // NC) + (s >> 1)
    h = s & 1

    pltpu.sync_copy(mask_hbm.at[b], mask_v)

    @pl.when(h == 0)
    def _():
        pltpu.sync_copy(ind1_hbm.at[b], ind_v)
        pltpu.sync_copy(tag1_hbm.at[b], tagrow)

    @pl.when(h == 1)
    def _():
        pltpu.sync_copy(ind2_hbm.at[b], ind_v)
        pltpu.sync_copy(tag2_hbm.at[b], tagrow)

    for jc in range(NCHUNK):
        sl = pl.ds(jc * L, L)
        tmine[sl] = plsc.load_gather(tagrow, [ind_v[sl]])

    pltpu.sync_copy(tmine, shared.at[s])
    plsc.subcore_barrier()
    pltpu.sync_copy(shared.at[s ^ 1], tother)

    inf_v = jnp.full((L,), jnp.inf, jnp.float32)
    for jc in range(NCHUNK):
        mcomp[pl.ds(jc * L, L)] = inf_v
    offset = jnp.int32(0)
    psum = jnp.zeros((L,), jnp.float32)
    for jc in range(NCHUNK):
        sl = pl.ds(jc * L, L)
        m = mask_v[sl]
        mb = m > 0
        a = tmine[sl]
        bb = tother[sl]
        mean = (a + bb) * 0.5
        d = a - bb
        psum = psum + jnp.where(mb, d * d, 0.0)
        pos = jnp.maximum(offset + lax.cumsum(m, axis=0) - 1, 0)
        plsc.store_scatter(mcomp, [pos], mean, mask=mb)
        offset = offset + jnp.sum(m)
    n = offset
    pullsq = jnp.sum(psum)

    half = (n + 1) >> 1
    i_lo = h * half
    i_hi = jnp.where(h == 0, half, n)
    mj = [mcomp[pl.ds(jc * L, L)] for jc in range(NCHUNK)]
    zero = jnp.zeros((L,), jnp.float32)

    def body(i, accs):
        accl = list(accs)
        miv = plsc.load_gather(mcomp, [jnp.full((L,), i, jnp.int32)])
        for jc in range(NCHUNK):
            r = jnp.maximum(1.0 - jnp.abs(mj[jc] - miv), 0.0)
            accl[jc % NACC] = accl[jc % NACC] + r
        return tuple(accl)

    accs = lax.fori_loop(i_lo, i_hi, body, (zero,) * NACC)
    S = jnp.sum(sum(accs[1:], accs[0]))

    h0 = h == 0
    nfv = jnp.full((L,), n.astype(jnp.float32))
    pullv = jnp.full((L,), jnp.where(h0, pullsq, 0.0)) / (2.0 * (nfv + 1e-4))
    corr = jnp.where(h0, nfv * nfv / (nfv + 1e-4), 0.0)
    pushv = (jnp.full((L,), S) - corr) / ((nfv - 1.0) * nfv + 1e-4)
    lane = lax.iota(jnp.int32, L)
    row_v[...] = jnp.where(lane == 0, pullv,
                           jnp.where(lane == 1, pushv, 0.0))
    pltpu.sync_copy(row_v, out_hbm.at[c * NS + s])


@functools.partial(
    pl.kernel,
    out_type=jax.ShapeDtypeStruct((NC * NS, L), jnp.float32),
    mesh=plsc.VectorSubcoreMesh(core_axis_name="c", subcore_axis_name="s"),
    compiler_params=pltpu.CompilerParams(needs_layout_passes=False),
    scratch_types=[
        pltpu.VMEM((HW,), jnp.float32),
        pltpu.VMEM((KP,), jnp.int32),
        pltpu.VMEM((KP,), jnp.int32),
        pltpu.VMEM((KP,), jnp.float32),
        pltpu.VMEM((KP,), jnp.float32),
        pltpu.VMEM((KP,), jnp.float32),
        pltpu.VMEM((L,), jnp.float32),
        pltpu.VMEM_SHARED((NS, KP), jnp.float32),
    ],
)
def _tag_loss_sc(tag1_hbm, tag2_hbm, ind1_hbm, ind2_hbm, mask_hbm, out_hbm,
                 *scratch):
    _tec_body(tag1_hbm, tag2_hbm, ind1_hbm, ind2_hbm, mask_hbm, out_hbm,
              *scratch)


@jax.jit
def kernel(tag1, tag2, ind1, ind2, mask):
    tag1f = tag1.reshape(B, HW)
    tag2f = tag2.reshape(B, HW)
    pad = ((0, 0), (0, KP - K))
    ind1p = jnp.pad(ind1.astype(jnp.int32), pad)
    ind2p = jnp.pad(ind2.astype(jnp.int32), pad)
    maskp = jnp.pad(mask.astype(jnp.int32), pad)
    out = _tag_loss_sc(tag1f, tag2f, ind1p, ind2p, maskp)
    return (out[:, 0].sum(), out[:, 1].sum())


# speedup vs baseline: 1.4152x; 1.0774x over previous
import functools

import jax
import jax.numpy as jnp
from jax import lax
from jax.experimental import pallas as pl
from jax.experimental.pallas import tpu as pltpu
from jax.experimental.pallas import tpu_sc as plsc

NC, NS, L = 2, 16, 16
B = 16
K = 500
KP = 512
NCHUNK = KP // L
HW = 128 * 128
NACC = 4


def _tec_body(tag1_hbm, tag2_hbm, ind1_hbm, ind2_hbm, mask_hbm, out_hbm,
              tagrow, ind_v, mask_v, tmine, tother, mcomp, row_v, shared):
    c = lax.axis_index("c")
    s = lax.axis_index("s")
    b = c * (B // NC) + (s >> 1)
    h = s & 1

    pltpu.sync_copy(mask_hbm.at[b], mask_v)

    @pl.when(h == 0)
    def _():
        pltpu.sync_copy(ind1_hbm.at[b], ind_v)
        pltpu.sync_copy(tag1_hbm.at[b], tagrow)

    @pl.when(h == 1)
    def _():
        pltpu.sync_copy(ind2_hbm.at[b], ind_v)
        pltpu.sync_copy(tag2_hbm.at[b], tagrow)

    for jc in range(NCHUNK):
        sl = pl.ds(jc * L, L)
        tmine[sl] = plsc.load_gather(tagrow, [ind_v[sl]])

    pltpu.sync_copy(tmine, shared.at[s])
    plsc.subcore_barrier()
    pltpu.sync_copy(shared.at[s ^ 1], tother)

    iota = lax.iota(jnp.int32, L)
    for jc in range(NCHUNK):
        mcomp[pl.ds(jc * L, L)] = (
            2.0e6 + 2.0 * (jc * L + iota).astype(jnp.float32))
    offset = jnp.int32(0)
    psum = jnp.zeros((L,), jnp.float32)
    for jc in range(NCHUNK):
        sl = pl.ds(jc * L, L)
        m = mask_v[sl]
        mb = m > 0
        a = tmine[sl]
        bb = tother[sl]
        mean = (a + bb) * 0.5
        d = a - bb
        psum = psum + jnp.where(mb, d * d, 0.0)
        pos = jnp.maximum(offset + lax.cumsum(m, axis=0) - 1, 0)
        plsc.store_scatter(mcomp, [pos], mean, mask=mb)
        offset = offset + jnp.sum(m)
    n = offset
    pullsq = jnp.sum(psum)

    nb = (n + L - 1) >> 4
    my_blocks = (nb - h + 1) >> 1
    zero = jnp.zeros((L,), jnp.float32)

    def outer(t, accs):
        rb = 2 * t + h
        base = rb * L
        basev = jnp.full((L,), base, jnp.int32)
        riv = plsc.load_gather(mcomp, [basev + iota])
        valid = (basev + iota) < n
        mis = [plsc.load_gather(mcomp, [jnp.full((L,), base + l, jnp.int32)])
               for l in range(L)]
        accl = list(accs)
        for l in range(L):
            r = jnp.maximum(1.0 - jnp.abs(riv - mis[l]), 0.0)
            accl[l % NACC] = accl[l % NACC] + jnp.where(
                (iota >= l) & valid, r, 0.0)

        def inner(jc, accs2):
            mjv = plsc.load_gather(
                mcomp, [jnp.full((L,), jc * L, jnp.int32) + iota])
            a2 = list(accs2)
            for l in range(L):
                r = jnp.maximum(1.0 - jnp.abs(mjv - mis[l]), 0.0)
                a2[l % NACC] = a2[l % NACC] + r
            return tuple(a2)

        return lax.fori_loop(rb + 1, nb, inner, tuple(accl))

    accs = lax.fori_loop(0, my_blocks, outer, (zero,) * NACC)
    Sp = jnp.sum(sum(accs[1:], accs[0]))

    h0 = h == 0
    nfv = jnp.full((L,), n.astype(jnp.float32))
    pullv = jnp.full((L,), jnp.where(h0, pullsq, 0.0)) / (2.0 * (nfv + 1e-4))
    corr = jnp.where(h0, nfv + nfv * nfv / (nfv + 1e-4), 0.0)
    pushv = (2.0 * jnp.full((L,), Sp) - corr) / ((nfv - 1.0) * nfv + 1e-4)
    lane = lax.iota(jnp.int32, L)
    row_v[...] = jnp.where(lane == 0, pullv,
                           jnp.where(lane == 1, pushv, 0.0))
    pltpu.sync_copy(row_v, out_hbm.at[c * NS + s])


@functools.partial(
    pl.kernel,
    out_type=jax.ShapeDtypeStruct((NC * NS, L), jnp.float32),
    mesh=plsc.VectorSubcoreMesh(core_axis_name="c", subcore_axis_name="s"),
    compiler_params=pltpu.CompilerParams(needs_layout_passes=False),
    scratch_types=[
        pltpu.VMEM((HW,), jnp.float32),
        pltpu.VMEM((KP,), jnp.int32),
        pltpu.VMEM((KP,), jnp.int32),
        pltpu.VMEM((KP,), jnp.float32),
        pltpu.VMEM((KP,), jnp.float32),
        pltpu.VMEM((KP,), jnp.float32),
        pltpu.VMEM((L,), jnp.float32),
        pltpu.VMEM_SHARED((NS, KP), jnp.float32),
    ],
)
def _tag_loss_sc(tag1_hbm, tag2_hbm, ind1_hbm, ind2_hbm, mask_hbm, out_hbm,
                 *scratch):
    _tec_body(tag1_hbm, tag2_hbm, ind1_hbm, ind2_hbm, mask_hbm, out_hbm,
              *scratch)


@jax.jit
def kernel(tag1, tag2, ind1, ind2, mask):
    tag1f = tag1.reshape(B, HW)
    tag2f = tag2.reshape(B, HW)
    pad = ((0, 0), (0, KP - K))
    ind1p = jnp.pad(ind1.astype(jnp.int32), pad)
    ind2p = jnp.pad(ind2.astype(jnp.int32), pad)
    maskp = jnp.pad(mask.astype(jnp.int32), pad)
    out = _tag_loss_sc(tag1f, tag2f, ind1p, ind2p, maskp)
    return (out[:, 0].sum(), out[:, 1].sum())
